# Initial kernel scaffold; baseline (speedup 1.0000x reference)
import jax
import jax.numpy as jnp
from jax.experimental import pallas as pl


def kernel(x, edge_index, batch, W1, b1, p1, W2, b2, p2, W3, b3, p3, lw1, lb1, lw2, lb2, lw3, lb3):
    raise NotImplementedError("write your pallas kernel here")


# speedup vs baseline: 2.2681x; 2.2681x over previous
import jax
import jax.numpy as jnp
from jax import lax
from jax.experimental import pallas as pl

_N = 100000
_G = 64
_BLK = 1000
_NBLK = _N // _BLK


def _mm_kernel(x_ref, w_ref, o_ref):
    o_ref[...] = jnp.dot(x_ref[...], w_ref[...],
                         preferred_element_type=jnp.float32)


def _mm(x, w):
    d_in, d_out = w.shape
    return pl.pallas_call(
        _mm_kernel,
        grid=(_NBLK,),
        in_specs=[pl.BlockSpec((_BLK, d_in), lambda i: (i, 0)),
                  pl.BlockSpec((d_in, d_out), lambda i: (0, 0))],
        out_specs=pl.BlockSpec((_BLK, d_out), lambda i: (i, 0)),
        out_shape=jax.ShapeDtypeStruct((x.shape[0], d_out), jnp.float32),
    )(x, w)


def _stage1_kernel(s_ref, nm_ref, lin_ref, q_ref, dinv_ref):
    nm = nm_ref[...]
    deg = nm * (s_ref[...] + 1.0)
    dinv = jnp.where(deg > 0.0, lax.rsqrt(jnp.maximum(deg, 1e-30)), 0.0)
    dinv_ref[...] = dinv
    q_ref[...] = lin_ref[...] * (dinv * nm)


def _stage1(s_col, nm_col, lin):
    return pl.pallas_call(
        _stage1_kernel,
        grid=(_NBLK,),
        in_specs=[pl.BlockSpec((_BLK, 1), lambda i: (i, 0)),
                  pl.BlockSpec((_BLK, 1), lambda i: (i, 0)),
                  pl.BlockSpec((_BLK, 64), lambda i: (i, 0))],
        out_specs=[pl.BlockSpec((_BLK, 64), lambda i: (i, 0)),
                   pl.BlockSpec((_BLK, 1), lambda i: (i, 0))],
        out_shape=[jax.ShapeDtypeStruct((_N, 64), jnp.float32),
                   jax.ShapeDtypeStruct((_N, 1), jnp.float32)],
    )(s_col, nm_col, lin)


def _stage2_kernel(agg_ref, dinv_ref, nm_ref, lin_ref, b_ref, o_ref):
    nm = nm_ref[...]
    dinv = dinv_ref[...]
    out = agg_ref[...] * (dinv * nm) + lin_ref[...] * (dinv * dinv * nm)
    o_ref[...] = jnp.maximum((out + b_ref[...]) * nm, 0.0)


def _stage2(agg, dinv_col, nm_col, lin, b):
    return pl.pallas_call(
        _stage2_kernel,
        grid=(_NBLK,),
        in_specs=[pl.BlockSpec((_BLK, 64), lambda i: (i, 0)),
                  pl.BlockSpec((_BLK, 1), lambda i: (i, 0)),
                  pl.BlockSpec((_BLK, 1), lambda i: (i, 0)),
                  pl.BlockSpec((_BLK, 64), lambda i: (i, 0)),
                  pl.BlockSpec((1, 64), lambda i: (0, 0))],
        out_specs=pl.BlockSpec((_BLK, 64), lambda i: (i, 0)),
        out_shape=jax.ShapeDtypeStruct((_N, 64), jnp.float32),
    )(agg, dinv_col, nm_col, lin, b)


def _mlp_kernel(z_ref, lw1_ref, lb1_ref, lw2_ref, lb2_ref, lw3_ref, lb3_ref,
                o_ref):
    z = z_ref[...]
    h1 = jnp.maximum(jnp.dot(z, lw1_ref[...],
                             preferred_element_type=jnp.float32)
                     + lb1_ref[...], 0.0)
    h2 = jnp.maximum(jnp.dot(h1, lw2_ref[...],
                             preferred_element_type=jnp.float32)
                     + lb2_ref[...], 0.0)
    z3 = jnp.dot(h2, lw3_ref[...], preferred_element_type=jnp.float32) \
        + lb3_ref[...]
    o_ref[...] = 1.0 / (1.0 + jnp.exp(-z3))


def _topk_j(x, p, batch, nmask, ratio, num_graphs):
    n = x.shape[0]
    score = (x @ p) / jnp.sqrt(jnp.sum(p * p))
    score_m = jnp.where(nmask > 0, score, -1e30)
    order = jnp.lexsort((-score_m, batch))
    pos = jnp.zeros((n,), jnp.int32).at[order].set(
        jnp.arange(n, dtype=jnp.int32))
    total = jax.ops.segment_sum(jnp.ones((n,), jnp.float32), batch,
                                num_segments=num_graphs)
    starts = jnp.cumsum(total) - total
    rank = pos.astype(jnp.float32) - starts[batch]
    nkept = jax.ops.segment_sum(nmask, batch, num_segments=num_graphs)
    k = jnp.ceil(ratio * nkept)
    keep = jnp.where(rank < k[batch], 1.0, 0.0) * nmask
    newx = x * jnp.tanh(score)[:, None] * keep[:, None]
    return newx, keep


def _readout_j(x, batch, nmask, num_graphs):
    xm = jnp.where(nmask[:, None] > 0, x, -1e30)
    gmp = jax.ops.segment_max(xm, batch, num_segments=num_graphs)
    s = jax.ops.segment_sum(x * nmask[:, None], batch,
                            num_segments=num_graphs)
    cnt = jax.ops.segment_sum(nmask, batch, num_segments=num_graphs)
    gap = s / jnp.maximum(cnt, 1.0)[:, None]
    return jnp.concatenate([gmp, gap], axis=1)


def _layer(h, W, b, p, src, dst, nm_col, batch):
    nmask = nm_col[:, 0]
    lin = _mm(h, W)
    s_col = jax.ops.segment_sum(nmask[src], dst, num_segments=_N)[:, None]
    q, dinv_col = _stage1(s_col, nm_col, lin)
    agg = jax.ops.segment_sum(q[src], dst, num_segments=_N)
    h = _stage2(agg, dinv_col, nm_col, lin, b[None, :])
    h, nmask = _topk_j(h, p, batch, nmask, 0.8, _G)
    return h, nmask, _readout_j(h, batch, nmask, _G)


def kernel(x, edge_index, batch, W1, b1, p1, W2, b2, p2, W3, b3, p3,
           lw1, lb1, lw2, lb2, lw3, lb3):
    src = edge_index[0]
    dst = edge_index[1]
    h = x[:, 0, :]
    nm_col = jnp.ones((_N, 1), jnp.float32)

    h, nmask, x1 = _layer(h, W1, b1, p1, src, dst, nm_col, batch)
    h, nmask, x2 = _layer(h, W2, b2, p2, src, dst, nmask[:, None], batch)
    h, nmask, x3 = _layer(h, W3, b3, p3, src, dst, nmask[:, None], batch)

    z = x1 + x2 + x3
    out = pl.pallas_call(
        _mlp_kernel,
        out_shape=jax.ShapeDtypeStruct((_G, 1), jnp.float32),
    )(z, lw1, lb1[None, :], lw2, lb2[None, :], lw3, lb3[None, :])
    return out[:, 0]
